# baseline (speedup 1.0000x reference)
import functools
import jax
import jax.numpy as jnp
from jax import lax
from jax.experimental import pallas as pl
from jax.experimental.pallas import tpu as pltpu
from jax.experimental.pallas import tpu_sc as plsc

N = 10000
E = 320000
G = 64
NPAD = 10240
NC, NS = 2, 16
NW = NC * NS
ROWS_PER_TILE = NPAD // NS
CHUNK = 128
E_PAD = 327680
EPT = E_PAD // NW
NBUF = 2

_mesh = plsc.VectorSubcoreMesh(core_axis_name="c", subcore_axis_name="s")


def _deg_body(ei_ref, ones_ref, zeros_ref, out_ref, hist, ones_v, idx_v, sem):
    cid = lax.axis_index("c")
    sid = lax.axis_index("s")
    r0 = sid * ROWS_PER_TILE
    pltpu.sync_copy(zeros_ref.at[pl.ds(r0, ROWS_PER_TILE)],
                    hist.at[pl.ds(r0, ROWS_PER_TILE)])
    pltpu.sync_copy(ones_ref, ones_v)
    plsc.subcore_barrier()

    n_chunks = E_PAD // NS // CHUNK
    pltpu.sync_copy(ei_ref.at[cid, sid], idx_v)

    W = 10

    @pl.loop(0, n_chunks, step=W)
    def _(g):
        for b in range(W):
            pltpu.async_copy(ones_v, hist.at[idx_v.at[g + b]], sem.at[b],
                             add=True)
        for b in range(W):
            pltpu.make_async_copy(ones_v, hist.at[idx_v.at[g + b]],
                                  sem.at[b]).wait()

    plsc.subcore_barrier()
    pltpu.sync_copy(hist.at[pl.ds(r0, ROWS_PER_TILE)],
                    out_ref.at[cid, pl.ds(r0, ROWS_PER_TILE)])


@jax.jit
def _degrees(ei4):
    ones = jnp.ones((CHUNK,), jnp.float32)
    zeros = jnp.zeros((NPAD,), jnp.float32)
    k = pl.kernel(
        _deg_body,
        out_type=jax.ShapeDtypeStruct((NC, NPAD), jnp.float32),
        mesh=_mesh,
        scratch_types=[
            pltpu.VMEM_SHARED((NPAD,), jnp.float32),
            pltpu.VMEM((CHUNK,), jnp.float32),
            pltpu.VMEM((E_PAD // NS // CHUNK, CHUNK), jnp.int32),
            pltpu.SemaphoreType.DMA((10,)),
        ],
    )
    return k(ei4, ones, zeros)


def _prop_body(p_ref, ei_ref, zeros_ref, out_ref,
               agg, sidx_v, didx_v, rows_v, gsem, ssem):
    cid = lax.axis_index("c")
    sid = lax.axis_index("s")
    r0 = sid * ROWS_PER_TILE
    pltpu.sync_copy(zeros_ref.at[pl.ds(r0, ROWS_PER_TILE)],
                    agg.at[pl.ds(r0, ROWS_PER_TILE)])
    wid = cid * NS + sid
    pltpu.sync_copy(ei_ref.at[0, wid], sidx_v)
    plsc.subcore_barrier()

    n_chunks = EPT // CHUNK

    def gat(j, b):
        return pltpu.make_async_copy(p_ref.at[sidx_v.at[j]], rows_v.at[b],
                                     gsem.at[b])

    def dix(j, b):
        return pltpu.make_async_copy(ei_ref.at[1, wid, j], didx_v.at[b],
                                     gsem.at[b])

    def sca(b):
        return pltpu.make_async_copy(rows_v.at[b], agg.at[didx_v.at[b]],
                                     ssem.at[b])

    def prefetch(j, b):
        pltpu.async_copy(p_ref.at[sidx_v.at[j]], rows_v.at[b], gsem.at[b])
        pltpu.async_copy(ei_ref.at[1, wid, j], didx_v.at[b], gsem.at[b])

    for b in range(NBUF):
        prefetch(b, b)

    @pl.loop(0, n_chunks - NBUF, step=NBUF)
    def _(g):
        for b in range(NBUF):
            gat(g + b, b).wait()
            dix(g + b, b).wait()
            pltpu.async_copy(rows_v.at[b], agg.at[didx_v.at[b]],
                             ssem.at[b], add=True)
        for b in range(NBUF):
            sca(b).wait()
            prefetch(g + b + NBUF, b)

    g0 = n_chunks - NBUF
    for b in range(NBUF):
        gat(g0 + b, b).wait()
        dix(g0 + b, b).wait()
        pltpu.async_copy(rows_v.at[b], agg.at[didx_v.at[b]],
                         ssem.at[b], add=True)
    for b in range(NBUF):
        sca(b).wait()

    plsc.subcore_barrier()
    pltpu.sync_copy(agg.at[pl.ds(r0, ROWS_PER_TILE)],
                    out_ref.at[cid, pl.ds(r0, ROWS_PER_TILE)])


@functools.partial(jax.jit, static_argnames=("d",))
def _propagate(p, ei3, d):
    zeros = jnp.zeros((NPAD, d), jnp.float32)
    k = pl.kernel(
        _prop_body,
        out_type=jax.ShapeDtypeStruct((NC, NPAD, d), jnp.float32),
        mesh=_mesh,
        scratch_types=[
            pltpu.VMEM_SHARED((NPAD, d), jnp.float32),
            pltpu.VMEM((EPT // CHUNK, CHUNK), jnp.int32),
            pltpu.VMEM((NBUF, CHUNK), jnp.int32),
            pltpu.VMEM((NBUF, CHUNK, d), jnp.float32),
            pltpu.SemaphoreType.DMA((NBUF,)),
            pltpu.SemaphoreType.DMA((NBUF,)),
        ],
        compiler_params=pltpu.CompilerParams(use_tc_tiling_on_sc=False),
    )
    return k(p, ei3, zeros)


def _pre1_body(x_ref, deg_ref, w_ref, o_ref):
    norm = lax.rsqrt(jnp.maximum(deg_ref[...], 1.0))
    o_ref[...] = jnp.dot(x_ref[...] * norm, w_ref[...],
                         preferred_element_type=jnp.float32,
                         precision=lax.Precision.HIGHEST)


def _mid_body(agg_ref, din_ref, dout_ref, b1_ref, w2_ref, o_ref):
    ndst = lax.rsqrt(jnp.maximum(din_ref[...], 1.0))
    nsrc = lax.rsqrt(jnp.maximum(dout_ref[...], 1.0))
    a = agg_ref[0, :N, :] + agg_ref[1, :N, :]
    h1 = jnp.maximum(a * ndst + b1_ref[...], 0.0)
    o_ref[...] = jnp.dot(h1 * nsrc, w2_ref[...],
                         preferred_element_type=jnp.float32,
                         precision=lax.Precision.HIGHEST)


def _post_body(agg_ref, din_ref, b2_ref, gid_ref,
               wc1_ref, bc1_ref, wc2_ref, bc2_ref, wc3_ref, bc3_ref, o_ref):
    ndst = lax.rsqrt(jnp.maximum(din_ref[...], 1.0))
    a = agg_ref[0, :N, :] + agg_ref[1, :N, :]
    h2 = jnp.maximum(a * ndst + b2_ref[...], 0.0)
    gid = gid_ref[...]
    rows = lax.broadcasted_iota(jnp.int32, (G, N), 0)
    onehot = (rows == gid).astype(jnp.float32)
    sums = jnp.dot(onehot, h2, preferred_element_type=jnp.float32,
                   precision=lax.Precision.HIGHEST)
    counts = jnp.sum(onehot, axis=1, keepdims=True)
    hg = sums / jnp.maximum(counts, 1.0)
    out = jnp.dot(hg, wc1_ref[...], preferred_element_type=jnp.float32,
                  precision=lax.Precision.HIGHEST) + bc1_ref[...]
    out = jnp.dot(out, wc2_ref[...], preferred_element_type=jnp.float32,
                  precision=lax.Precision.HIGHEST) + bc2_ref[...]
    out = jnp.dot(out, wc3_ref[...], preferred_element_type=jnp.float32,
                  precision=lax.Precision.HIGHEST) + bc3_ref[...]
    o_ref[...] = out


def _tc_call(body, out_shape, *args):
    return pl.pallas_call(body, out_shape=out_shape)(*args)


def kernel(x, edge_index, graph_ids, W1, b1, W2, b2,
           Wc1, bc1, Wc2, bc2, Wc3, bc3):
    ei = edge_index.astype(jnp.int32)
    npad_edges = E_PAD - E
    ei_deg = jnp.pad(ei, ((0, 0), (0, npad_edges)), constant_values=NPAD - 1)
    ei_deg = ei_deg.reshape(2, NS, E_PAD // NS // CHUNK, CHUNK)
    pad_src = jnp.zeros((npad_edges,), jnp.int32)
    pad_dst = jnp.full((npad_edges,), NPAD - 1, jnp.int32)
    ei_prop = jnp.concatenate(
        [ei, jnp.stack([pad_src, pad_dst])], axis=1
    ).reshape(2, NW, EPT // CHUNK, CHUNK)

    degs = _degrees(ei_deg)
    deg_out = degs[0, :N].reshape(N, 1)
    deg_in = degs[1, :N].reshape(N, 1)

    p1 = _tc_call(_pre1_body, jax.ShapeDtypeStruct((N, W1.shape[1]), jnp.float32),
                  x, deg_out, W1)

    agg1 = _propagate(p1, ei_prop, W1.shape[1])
    p2 = _tc_call(_mid_body, jax.ShapeDtypeStruct((N, W2.shape[1]), jnp.float32),
                  agg1, deg_in, deg_out, b1.reshape(1, -1), W2)

    agg2 = _propagate(p2, ei_prop, W2.shape[1])
    out = _tc_call(_post_body,
                   jax.ShapeDtypeStruct((G, Wc3.shape[1]), jnp.float32),
                   agg2, deg_in, b2.reshape(1, -1),
                   graph_ids.astype(jnp.int32).reshape(1, N),
                   Wc1, bc1.reshape(1, -1), Wc2, bc2.reshape(1, -1),
                   Wc3, bc3.reshape(1, -1))
    return out


# speedup vs baseline: 2.6835x; 2.6835x over previous
import functools
import jax
import jax.numpy as jnp
from jax import lax
from jax.experimental import pallas as pl
from jax.experimental.pallas import tpu as pltpu
from jax.experimental.pallas import tpu_sc as plsc

N = 10000
E = 320000
G = 64
NPAD = 10240
NC, NS = 2, 16
NW = NC * NS
ROWS_PER_TILE = NPAD // NS
CHUNK = 128
EPT = E // NW
NFULL = EPT // CHUNK
TAIL = EPT - NFULL * CHUNK
NBUF = 2
DEPT = E // NS
DFULL = DEPT // CHUNK
DTAIL = DEPT - DFULL * CHUNK
DW = 12

_mesh = plsc.VectorSubcoreMesh(core_axis_name="c", subcore_axis_name="s")


def _deg_body(src_ref, dst_ref, ones_ref, zeros_ref, out_ref,
              hist, ones_v, idx_v, tidx_v, gsem, ssem):
    cid = lax.axis_index("c")
    sid = lax.axis_index("s")
    r0 = sid * ROWS_PER_TILE
    pltpu.sync_copy(zeros_ref.at[pl.ds(r0, ROWS_PER_TILE)],
                    hist.at[pl.ds(r0, ROWS_PER_TILE)])
    pltpu.sync_copy(ones_ref, ones_v)
    plsc.subcore_barrier()

    e0 = sid * DEPT

    def run(e_ref):
        def idx(j, b):
            return pltpu.make_async_copy(
                e_ref.at[pl.ds(e0 + j * CHUNK, CHUNK)], idx_v.at[b],
                gsem.at[b])

        def sca(b):
            return pltpu.make_async_copy(ones_v, hist.at[idx_v.at[b]],
                                         ssem.at[b])

        for b in range(DW):
            idx(b, b).start()

        @pl.loop(0, DFULL - DW, step=DW)
        def _(g):
            for b in range(DW):
                idx(g + b, b).wait()
                pltpu.async_copy(ones_v, hist.at[idx_v.at[b]], ssem.at[b],
                                 add=True)
            for b in range(DW):
                sca(b).wait()
                idx(g + b + DW, b).start()

        g0 = DFULL - DW
        for b in range(DW):
            idx(g0 + b, b).wait()
            pltpu.async_copy(ones_v, hist.at[idx_v.at[b]], ssem.at[b],
                             add=True)
        for b in range(DW):
            sca(b).wait()

        pltpu.sync_copy(e_ref.at[pl.ds(e0 + DFULL * CHUNK, DTAIL)], tidx_v)
        pltpu.sync_copy(ones_v.at[pl.ds(0, DTAIL)], hist.at[tidx_v],
                        add=True)

    @pl.when(cid == 0)
    def _():
        run(src_ref)

    @pl.when(cid == 1)
    def _():
        run(dst_ref)

    plsc.subcore_barrier()
    pltpu.sync_copy(hist.at[pl.ds(r0, ROWS_PER_TILE)],
                    out_ref.at[cid, pl.ds(r0, ROWS_PER_TILE)])


@jax.jit
def _degrees(src, dst):
    ones = jnp.ones((CHUNK,), jnp.float32)
    zeros = jnp.zeros((NPAD,), jnp.float32)
    k = pl.kernel(
        _deg_body,
        out_type=jax.ShapeDtypeStruct((NC, NPAD), jnp.float32),
        mesh=_mesh,
        scratch_types=[
            pltpu.VMEM_SHARED((NPAD,), jnp.float32),
            pltpu.VMEM((CHUNK,), jnp.float32),
            pltpu.VMEM((DW, CHUNK), jnp.int32),
            pltpu.VMEM((DTAIL,), jnp.int32),
            pltpu.SemaphoreType.DMA((DW,)),
            pltpu.SemaphoreType.DMA((DW,)),
        ],
    )
    return k(src, dst, ones, zeros)


def _prop_body(p_ref, src_ref, dst_ref, zeros_ref, out_ref,
               agg, sidx_v, didx_v, tdidx_v, rows_v, gsem, ssem):
    cid = lax.axis_index("c")
    sid = lax.axis_index("s")
    r0 = sid * ROWS_PER_TILE
    pltpu.sync_copy(zeros_ref.at[pl.ds(r0, ROWS_PER_TILE)],
                    agg.at[pl.ds(r0, ROWS_PER_TILE)])
    wid = cid * NS + sid
    e0 = wid * EPT
    pltpu.sync_copy(src_ref.at[pl.ds(e0, EPT)], sidx_v)
    plsc.subcore_barrier()

    def gat(j, b):
        return pltpu.make_async_copy(
            p_ref.at[sidx_v.at[pl.ds(j * CHUNK, CHUNK)]], rows_v.at[b],
            gsem.at[b])

    def dix(j, b):
        return pltpu.make_async_copy(
            dst_ref.at[pl.ds(e0 + j * CHUNK, CHUNK)], didx_v.at[b],
            gsem.at[b])

    def sca(b):
        return pltpu.make_async_copy(rows_v.at[b], agg.at[didx_v.at[b]],
                                     ssem.at[b])

    def prefetch(j, b):
        gat(j, b).start()
        dix(j, b).start()

    for b in range(NBUF):
        prefetch(b, b)

    @pl.loop(0, NFULL - NBUF, step=NBUF)
    def _(g):
        for b in range(NBUF):
            gat(g + b, b).wait()
            dix(g + b, b).wait()
            pltpu.async_copy(rows_v.at[b], agg.at[didx_v.at[b]],
                             ssem.at[b], add=True)
        for b in range(NBUF):
            sca(b).wait()
            prefetch(g + b + NBUF, b)

    g0 = NFULL - NBUF
    for b in range(NBUF):
        gat(g0 + b, b).wait()
        dix(g0 + b, b).wait()
        pltpu.async_copy(rows_v.at[b], agg.at[didx_v.at[b]],
                         ssem.at[b], add=True)
    for b in range(NBUF):
        sca(b).wait()

    et = e0 + NFULL * CHUNK
    pltpu.sync_copy(dst_ref.at[pl.ds(et, TAIL)], tdidx_v)
    trows = rows_v.at[0, pl.ds(0, TAIL)]
    pltpu.async_copy(
        p_ref.at[sidx_v.at[pl.ds(NFULL * CHUNK, TAIL)]], trows,
        gsem.at[0]).wait()
    pltpu.sync_copy(trows, agg.at[tdidx_v], add=True)

    plsc.subcore_barrier()
    pltpu.sync_copy(agg.at[pl.ds(r0, ROWS_PER_TILE)],
                    out_ref.at[cid, pl.ds(r0, ROWS_PER_TILE)])


@functools.partial(jax.jit, static_argnames=("d",))
def _propagate(p, src, dst, d):
    zeros = jnp.zeros((NPAD, d), jnp.float32)
    cp = pltpu.CompilerParams(use_tc_tiling_on_sc=(d % 128 == 0))
    k = pl.kernel(
        _prop_body,
        out_type=jax.ShapeDtypeStruct((NC, NPAD, d), jnp.float32),
        mesh=_mesh,
        scratch_types=[
            pltpu.VMEM_SHARED((NPAD, d), jnp.float32),
            pltpu.VMEM((EPT,), jnp.int32),
            pltpu.VMEM((NBUF, CHUNK), jnp.int32),
            pltpu.VMEM((TAIL,), jnp.int32),
            pltpu.VMEM((NBUF, CHUNK, d), jnp.float32),
            pltpu.SemaphoreType.DMA((NBUF,)),
            pltpu.SemaphoreType.DMA((NBUF,)),
        ],
        compiler_params=cp,
    )
    return k(p, src, dst, zeros)


RB = 2000


def _hi_dot(a, b):
    return jnp.dot(a, b, preferred_element_type=jnp.float32,
                   precision=lax.Precision.HIGHEST)


def _pre1_body(x_ref, deg_ref, w_ref, o_ref):
    norm = lax.rsqrt(jnp.maximum(deg_ref[...], 1.0))
    o_ref[...] = _hi_dot(x_ref[...] * norm, w_ref[...])


def _mid_body(agg_ref, din_ref, dout_ref, b1_ref, w2_ref, o_ref):
    ndst = lax.rsqrt(jnp.maximum(din_ref[...], 1.0))
    nsrc = lax.rsqrt(jnp.maximum(dout_ref[...], 1.0))
    a = agg_ref[0] + agg_ref[1]
    h1 = jnp.maximum(a * ndst + b1_ref[...], 0.0)
    o_ref[...] = _hi_dot(h1 * nsrc, w2_ref[...])


def _post_body(agg_ref, din_ref, b2_ref, gid_ref,
               wc1_ref, bc1_ref, wc2_ref, bc2_ref, wc3_ref, bc3_ref,
               o_ref, sums_acc, counts_acc):
    i = pl.program_id(0)
    ndst = lax.rsqrt(jnp.maximum(din_ref[...], 1.0))
    a = agg_ref[0] + agg_ref[1]
    h2 = jnp.maximum(a * ndst + b2_ref[...], 0.0)
    gid = gid_ref[0]
    rows = lax.broadcasted_iota(jnp.int32, (G, RB), 0)
    onehot = (rows == gid).astype(jnp.float32)
    s = _hi_dot(onehot, h2)
    c = jnp.sum(onehot, axis=1, keepdims=True)

    @pl.when(i == 0)
    def _():
        sums_acc[...] = s
        counts_acc[...] = c

    @pl.when(i > 0)
    def _():
        sums_acc[...] += s
        counts_acc[...] += c

    @pl.when(i == N // RB - 1)
    def _():
        hg = sums_acc[...] / jnp.maximum(counts_acc[...], 1.0)
        out = _hi_dot(hg, wc1_ref[...]) + bc1_ref[...]
        out = _hi_dot(out, wc2_ref[...]) + bc2_ref[...]
        out = _hi_dot(out, wc3_ref[...]) + bc3_ref[...]
        o_ref[...] = out


def _row_spec(d):
    return pl.BlockSpec((RB, d), lambda i: (i, 0))


def _fix_spec(shape):
    nd = len(shape)
    return pl.BlockSpec(shape, lambda i: (0,) * nd)


def _pre1(x, deg_out, W1):
    return pl.pallas_call(
        _pre1_body,
        grid=(N // RB,),
        in_specs=[_row_spec(x.shape[1]), _row_spec(1), _fix_spec(W1.shape)],
        out_specs=_row_spec(W1.shape[1]),
        out_shape=jax.ShapeDtypeStruct((N, W1.shape[1]), jnp.float32),
    )(x, deg_out, W1)


def _mid(agg1, deg_in, deg_out, b1, W2):
    d = agg1.shape[2]
    return pl.pallas_call(
        _mid_body,
        grid=(N // RB,),
        in_specs=[pl.BlockSpec((NC, RB, d), lambda i: (0, i, 0)),
                  _row_spec(1), _row_spec(1),
                  _fix_spec((1, d)), _fix_spec(W2.shape)],
        out_specs=_row_spec(W2.shape[1]),
        out_shape=jax.ShapeDtypeStruct((N, W2.shape[1]), jnp.float32),
    )(agg1, deg_in, deg_out, b1, W2)


def _post(agg2, deg_in, b2, gid, Wc1, bc1, Wc2, bc2, Wc3, bc3):
    d = agg2.shape[2]
    return pl.pallas_call(
        _post_body,
        grid=(N // RB,),
        in_specs=[pl.BlockSpec((NC, RB, d), lambda i: (0, i, 0)),
                  _row_spec(1),
                  _fix_spec((1, d)),
                  pl.BlockSpec((1, 1, RB), lambda i: (i, 0, 0)),
                  _fix_spec(Wc1.shape), _fix_spec((1, Wc1.shape[1])),
                  _fix_spec(Wc2.shape), _fix_spec((1, Wc2.shape[1])),
                  _fix_spec(Wc3.shape), _fix_spec((1, Wc3.shape[1]))],
        out_specs=_fix_spec((G, Wc3.shape[1])),
        out_shape=jax.ShapeDtypeStruct((G, Wc3.shape[1]), jnp.float32),
        scratch_shapes=[pltpu.VMEM((G, d), jnp.float32),
                        pltpu.VMEM((G, 1), jnp.float32)],
    )(agg2, deg_in, b2, gid, Wc1, bc1, Wc2, bc2, Wc3, bc3)


def kernel(x, edge_index, graph_ids, W1, b1, W2, b2,
           Wc1, bc1, Wc2, bc2, Wc3, bc3):
    ei = edge_index.astype(jnp.int32)
    src = ei[0]
    dst = ei[1]

    degs = _degrees(src, dst)
    deg_out = degs[0, :N].reshape(N, 1)
    deg_in = degs[1, :N].reshape(N, 1)

    p1 = _pre1(x, deg_out, W1)
    agg1 = _propagate(p1, src, dst, W1.shape[1])
    p2 = _mid(agg1, deg_in, deg_out, b1.reshape(1, -1), W2)
    agg2 = _propagate(p2, src, dst, W2.shape[1])
    out = _post(agg2, deg_in, b2.reshape(1, -1),
                graph_ids.astype(jnp.int32).reshape(N // RB, 1, RB),
                Wc1, bc1.reshape(1, -1), Wc2, bc2.reshape(1, -1),
                Wc3, bc3.reshape(1, -1))
    return out


# speedup vs baseline: 2.8962x; 1.0793x over previous
import functools
import jax
import jax.numpy as jnp
from jax import lax
from jax.experimental import pallas as pl
from jax.experimental.pallas import tpu as pltpu
from jax.experimental.pallas import tpu_sc as plsc

N = 10000
E = 320000
G = 64
NPAD = 10240
NC, NS = 2, 16
NW = NC * NS
ROWS_PER_TILE = NPAD // NS
CHUNK = 128
EPT = E // NW
PCHUNK = 40
PCHUNKS = EPT // PCHUNK
PNBUF = 5
DEPT = E // NS
DFULL = DEPT // CHUNK
DTAIL = DEPT - DFULL * CHUNK
DW = 12

_mesh = plsc.VectorSubcoreMesh(core_axis_name="c", subcore_axis_name="s")


def _deg_body(src_ref, dst_ref, ones_ref, zeros_ref, out_ref,
              hist, ones_v, idx_v, tidx_v, gsem, ssem):
    cid = lax.axis_index("c")
    sid = lax.axis_index("s")
    r0 = sid * ROWS_PER_TILE
    pltpu.sync_copy(zeros_ref.at[pl.ds(r0, ROWS_PER_TILE)],
                    hist.at[pl.ds(r0, ROWS_PER_TILE)])
    pltpu.sync_copy(ones_ref, ones_v)
    plsc.subcore_barrier()

    e0 = sid * DEPT

    def run(e_ref):
        def idx(j, b):
            return pltpu.make_async_copy(
                e_ref.at[pl.ds(e0 + j * CHUNK, CHUNK)], idx_v.at[b],
                gsem.at[b])

        def sca(b):
            return pltpu.make_async_copy(ones_v, hist.at[idx_v.at[b]],
                                         ssem.at[b])

        for b in range(DW):
            idx(b, b).start()

        @pl.loop(0, DFULL - DW, step=DW)
        def _(g):
            for b in range(DW):
                idx(g + b, b).wait()
                pltpu.async_copy(ones_v, hist.at[idx_v.at[b]], ssem.at[b],
                                 add=True)
            for b in range(DW):
                sca(b).wait()
                idx(g + b + DW, b).start()

        g0 = DFULL - DW
        for b in range(DW):
            idx(g0 + b, b).wait()
            pltpu.async_copy(ones_v, hist.at[idx_v.at[b]], ssem.at[b],
                             add=True)
        for b in range(DW):
            sca(b).wait()

        pltpu.sync_copy(e_ref.at[pl.ds(e0 + DFULL * CHUNK, DTAIL)], tidx_v)
        pltpu.sync_copy(ones_v.at[pl.ds(0, DTAIL)], hist.at[tidx_v],
                        add=True)

    @pl.when(cid == 0)
    def _():
        run(src_ref)

    @pl.when(cid == 1)
    def _():
        run(dst_ref)

    plsc.subcore_barrier()
    pltpu.sync_copy(hist.at[pl.ds(r0, ROWS_PER_TILE)],
                    out_ref.at[cid, pl.ds(r0, ROWS_PER_TILE)])


@jax.jit
def _degrees(src, dst):
    ones = jnp.ones((CHUNK,), jnp.float32)
    zeros = jnp.zeros((NPAD,), jnp.float32)
    k = pl.kernel(
        _deg_body,
        out_type=jax.ShapeDtypeStruct((NC, NPAD), jnp.float32),
        mesh=_mesh,
        scratch_types=[
            pltpu.VMEM_SHARED((NPAD,), jnp.float32),
            pltpu.VMEM((CHUNK,), jnp.float32),
            pltpu.VMEM((DW, CHUNK), jnp.int32),
            pltpu.VMEM((DTAIL,), jnp.int32),
            pltpu.SemaphoreType.DMA((DW,)),
            pltpu.SemaphoreType.DMA((DW,)),
        ],
    )
    return k(src, dst, ones, zeros)


def _prop_body(p_ref, src_ref, dst_ref, zeros_ref, out_ref,
               agg, sidx_v, didx_v, rows_v, gsem, ssem):
    cid = lax.axis_index("c")
    sid = lax.axis_index("s")
    r0 = sid * ROWS_PER_TILE
    pltpu.sync_copy(zeros_ref.at[pl.ds(r0, ROWS_PER_TILE)],
                    agg.at[pl.ds(r0, ROWS_PER_TILE)])
    wid = cid * NS + sid
    e0 = wid * EPT
    pltpu.sync_copy(src_ref.at[pl.ds(e0, EPT)], sidx_v)
    plsc.subcore_barrier()

    def gat(j, b):
        return pltpu.make_async_copy(
            p_ref.at[sidx_v.at[pl.ds(j * PCHUNK, PCHUNK)]], rows_v.at[b],
            gsem.at[b])

    def dix(j, b):
        return pltpu.make_async_copy(
            dst_ref.at[pl.ds(e0 + j * PCHUNK, PCHUNK)], didx_v.at[b],
            gsem.at[b])

    def sca(b):
        return pltpu.make_async_copy(rows_v.at[b], agg.at[didx_v.at[b]],
                                     ssem.at[b])

    def prefetch(j, b):
        gat(j, b).start()
        dix(j, b).start()

    for b in range(PNBUF):
        prefetch(b, b)

    @pl.loop(0, PCHUNKS - PNBUF, step=PNBUF)
    def _(g):
        for b in range(PNBUF):
            gat(g + b, b).wait()
            dix(g + b, b).wait()
            pltpu.async_copy(rows_v.at[b], agg.at[didx_v.at[b]],
                             ssem.at[b], add=True)
        for b in range(PNBUF):
            sca(b).wait()
            prefetch(g + b + PNBUF, b)

    g0 = PCHUNKS - PNBUF
    for b in range(PNBUF):
        gat(g0 + b, b).wait()
        dix(g0 + b, b).wait()
        pltpu.async_copy(rows_v.at[b], agg.at[didx_v.at[b]],
                         ssem.at[b], add=True)
    for b in range(PNBUF):
        sca(b).wait()

    plsc.subcore_barrier()
    pltpu.sync_copy(agg.at[pl.ds(r0, ROWS_PER_TILE)],
                    out_ref.at[cid, pl.ds(r0, ROWS_PER_TILE)])


@functools.partial(jax.jit, static_argnames=("d",))
def _propagate(p, src, dst, d):
    zeros = jnp.zeros((NPAD, d), jnp.float32)
    cp = pltpu.CompilerParams(use_tc_tiling_on_sc=(d % 128 == 0))
    k = pl.kernel(
        _prop_body,
        out_type=jax.ShapeDtypeStruct((NC, NPAD, d), jnp.float32),
        mesh=_mesh,
        scratch_types=[
            pltpu.VMEM_SHARED((NPAD, d), jnp.float32),
            pltpu.VMEM((EPT,), jnp.int32),
            pltpu.VMEM((PNBUF, PCHUNK), jnp.int32),
            pltpu.VMEM((PNBUF, PCHUNK, d), jnp.float32),
            pltpu.SemaphoreType.DMA((PNBUF,)),
            pltpu.SemaphoreType.DMA((PNBUF,)),
        ],
        compiler_params=cp,
    )
    return k(p, src, dst, zeros)


RB = 2000
EB = 32000


def _split_body(ei_ref, src_ref, dst_ref):
    src_ref[...] = ei_ref[0:1]
    dst_ref[...] = ei_ref[1:2]


def _split(ei):
    src, dst = pl.pallas_call(
        _split_body,
        grid=(E // EB,),
        in_specs=[pl.BlockSpec((2, EB), lambda i: (0, i))],
        out_specs=[pl.BlockSpec((1, EB), lambda i: (0, i)),
                   pl.BlockSpec((1, EB), lambda i: (0, i))],
        out_shape=[jax.ShapeDtypeStruct((1, E), jnp.int32),
                   jax.ShapeDtypeStruct((1, E), jnp.int32)],
    )(ei)
    return src.reshape(E), dst.reshape(E)


def _hi_dot(a, b):
    return jnp.dot(a, b, preferred_element_type=jnp.float32,
                   precision=lax.Precision.HIGHEST)


def _pre1_body(x_ref, deg_ref, w_ref, o_ref):
    norm = lax.rsqrt(jnp.maximum(deg_ref[...], 1.0))
    o_ref[...] = _hi_dot(x_ref[...] * norm, w_ref[...])


def _mid_body(agg_ref, din_ref, dout_ref, b1_ref, w2_ref, o_ref):
    ndst = lax.rsqrt(jnp.maximum(din_ref[...], 1.0))
    nsrc = lax.rsqrt(jnp.maximum(dout_ref[...], 1.0))
    a = agg_ref[0] + agg_ref[1]
    h1 = jnp.maximum(a * ndst + b1_ref[...], 0.0)
    o_ref[...] = _hi_dot(h1 * nsrc, w2_ref[...])


def _post_body(agg_ref, din_ref, b2_ref, gid_ref,
               wc1_ref, bc1_ref, wc2_ref, bc2_ref, wc3_ref, bc3_ref,
               o_ref, sums_acc, counts_acc):
    i = pl.program_id(0)
    ndst = lax.rsqrt(jnp.maximum(din_ref[...], 1.0))
    a = agg_ref[0] + agg_ref[1]
    h2 = jnp.maximum(a * ndst + b2_ref[...], 0.0)
    gid = gid_ref[0]
    rows = lax.broadcasted_iota(jnp.int32, (G, RB), 0)
    onehot = (rows == gid).astype(jnp.float32)
    s = _hi_dot(onehot, h2)
    c = jnp.sum(onehot, axis=1, keepdims=True)

    @pl.when(i == 0)
    def _():
        sums_acc[...] = s
        counts_acc[...] = c

    @pl.when(i > 0)
    def _():
        sums_acc[...] += s
        counts_acc[...] += c

    @pl.when(i == N // RB - 1)
    def _():
        hg = sums_acc[...] / jnp.maximum(counts_acc[...], 1.0)
        out = _hi_dot(hg, wc1_ref[...]) + bc1_ref[...]
        out = _hi_dot(out, wc2_ref[...]) + bc2_ref[...]
        out = _hi_dot(out, wc3_ref[...]) + bc3_ref[...]
        o_ref[...] = out


def _row_spec(d):
    return pl.BlockSpec((RB, d), lambda i: (i, 0))


def _fix_spec(shape):
    nd = len(shape)
    return pl.BlockSpec(shape, lambda i: (0,) * nd)


def _pre1(x, deg_out, W1):
    return pl.pallas_call(
        _pre1_body,
        grid=(N // RB,),
        in_specs=[_row_spec(x.shape[1]), _row_spec(1), _fix_spec(W1.shape)],
        out_specs=_row_spec(W1.shape[1]),
        out_shape=jax.ShapeDtypeStruct((N, W1.shape[1]), jnp.float32),
    )(x, deg_out, W1)


def _mid(agg1, deg_in, deg_out, b1, W2):
    d = agg1.shape[2]
    return pl.pallas_call(
        _mid_body,
        grid=(N // RB,),
        in_specs=[pl.BlockSpec((NC, RB, d), lambda i: (0, i, 0)),
                  _row_spec(1), _row_spec(1),
                  _fix_spec((1, d)), _fix_spec(W2.shape)],
        out_specs=_row_spec(W2.shape[1]),
        out_shape=jax.ShapeDtypeStruct((N, W2.shape[1]), jnp.float32),
    )(agg1, deg_in, deg_out, b1, W2)


def _post(agg2, deg_in, b2, gid, Wc1, bc1, Wc2, bc2, Wc3, bc3):
    d = agg2.shape[2]
    return pl.pallas_call(
        _post_body,
        grid=(N // RB,),
        in_specs=[pl.BlockSpec((NC, RB, d), lambda i: (0, i, 0)),
                  _row_spec(1),
                  _fix_spec((1, d)),
                  pl.BlockSpec((1, 1, RB), lambda i: (i, 0, 0)),
                  _fix_spec(Wc1.shape), _fix_spec((1, Wc1.shape[1])),
                  _fix_spec(Wc2.shape), _fix_spec((1, Wc2.shape[1])),
                  _fix_spec(Wc3.shape), _fix_spec((1, Wc3.shape[1]))],
        out_specs=_fix_spec((G, Wc3.shape[1])),
        out_shape=jax.ShapeDtypeStruct((G, Wc3.shape[1]), jnp.float32),
        scratch_shapes=[pltpu.VMEM((G, d), jnp.float32),
                        pltpu.VMEM((G, 1), jnp.float32)],
    )(agg2, deg_in, b2, gid, Wc1, bc1, Wc2, bc2, Wc3, bc3)


def kernel(x, edge_index, graph_ids, W1, b1, W2, b2,
           Wc1, bc1, Wc2, bc2, Wc3, bc3):
    ei = edge_index.astype(jnp.int32)
    src, dst = _split(ei)

    degs = _degrees(src, dst)
    deg_out = degs[0, :N].reshape(N, 1)
    deg_in = degs[1, :N].reshape(N, 1)

    p1 = _pre1(x, deg_out, W1)
    agg1 = _propagate(p1, src, dst, W1.shape[1])
    p2 = _mid(agg1, deg_in, deg_out, b1.reshape(1, -1), W2)
    agg2 = _propagate(p2, src, dst, W2.shape[1])
    out = _post(agg2, deg_in, b2.reshape(1, -1),
                graph_ids.astype(jnp.int32).reshape(N // RB, 1, RB),
                Wc1, bc1.reshape(1, -1), Wc2, bc2.reshape(1, -1),
                Wc3, bc3.reshape(1, -1))
    return out


# speedup vs baseline: 3.2259x; 1.1139x over previous
import functools
import jax
import jax.numpy as jnp
from jax import lax
from jax.experimental import pallas as pl
from jax.experimental.pallas import tpu as pltpu
from jax.experimental.pallas import tpu_sc as plsc

N = 10000
E = 320000
G = 64
NPAD = 10240
NC, NS = 2, 16
NW = NC * NS
ROWS_PER_TILE = NPAD // NS
CHUNK = 128
EPT = E // NW
PCHUNK = 40
PCHUNKS = EPT // PCHUNK
PNBUF = 5
DEPT = E // NS
DFULL = DEPT // CHUNK
DTAIL = DEPT - DFULL * CHUNK
DW = 12

_mesh = plsc.VectorSubcoreMesh(core_axis_name="c", subcore_axis_name="s")


def _deg_body(src_ref, dst_ref, ones_ref, zeros_ref, out_ref,
              hist, ones_v, idx_v, tidx_v, gsem, ssem):
    cid = lax.axis_index("c")
    sid = lax.axis_index("s")
    r0 = sid * ROWS_PER_TILE
    pltpu.sync_copy(zeros_ref.at[pl.ds(r0, ROWS_PER_TILE)],
                    hist.at[pl.ds(r0, ROWS_PER_TILE)])
    pltpu.sync_copy(ones_ref, ones_v)
    plsc.subcore_barrier()

    e0 = sid * DEPT

    def run(e_ref):
        def idx(j, b):
            return pltpu.make_async_copy(
                e_ref.at[pl.ds(e0 + j * CHUNK, CHUNK)], idx_v.at[b],
                gsem.at[b])

        def sca(b):
            return pltpu.make_async_copy(ones_v, hist.at[idx_v.at[b]],
                                         ssem.at[b])

        for b in range(DW):
            idx(b, b).start()

        @pl.loop(0, DFULL - DW, step=DW)
        def _(g):
            for b in range(DW):
                idx(g + b, b).wait()
                pltpu.async_copy(ones_v, hist.at[idx_v.at[b]], ssem.at[b],
                                 add=True)
            for b in range(DW):
                sca(b).wait()
                idx(g + b + DW, b).start()

        g0 = DFULL - DW
        for b in range(DW):
            idx(g0 + b, b).wait()
            pltpu.async_copy(ones_v, hist.at[idx_v.at[b]], ssem.at[b],
                             add=True)
        for b in range(DW):
            sca(b).wait()

        pltpu.sync_copy(e_ref.at[pl.ds(e0 + DFULL * CHUNK, DTAIL)], tidx_v)
        pltpu.sync_copy(ones_v.at[pl.ds(0, DTAIL)], hist.at[tidx_v],
                        add=True)

    @pl.when(cid == 0)
    def _():
        run(src_ref)

    @pl.when(cid == 1)
    def _():
        run(dst_ref)

    plsc.subcore_barrier()
    pltpu.sync_copy(hist.at[pl.ds(r0, ROWS_PER_TILE)],
                    out_ref.at[cid, pl.ds(r0, ROWS_PER_TILE)])


@jax.jit
def _degrees(src, dst):
    ones = jnp.ones((CHUNK,), jnp.float32)
    zeros = jnp.zeros((NPAD,), jnp.float32)
    k = pl.kernel(
        _deg_body,
        out_type=jax.ShapeDtypeStruct((NC, NPAD), jnp.float32),
        mesh=_mesh,
        scratch_types=[
            pltpu.VMEM_SHARED((NPAD,), jnp.float32),
            pltpu.VMEM((CHUNK,), jnp.float32),
            pltpu.VMEM((DW, CHUNK), jnp.int32),
            pltpu.VMEM((DTAIL,), jnp.int32),
            pltpu.SemaphoreType.DMA((DW,)),
            pltpu.SemaphoreType.DMA((DW,)),
        ],
    )
    return k(src, dst, ones, zeros)


def _prop_body(p_ref, src_ref, dst_ref, zeros_ref, out_ref,
               agg, sidx_v, didx_v, rows_v, gsem, ssem):
    cid = lax.axis_index("c")
    sid = lax.axis_index("s")
    r0 = sid * ROWS_PER_TILE
    pltpu.sync_copy(zeros_ref.at[pl.ds(r0, ROWS_PER_TILE)],
                    agg.at[pl.ds(r0, ROWS_PER_TILE)])
    wid = cid * NS + sid
    e0 = wid * EPT
    pltpu.sync_copy(src_ref.at[pl.ds(e0, EPT)], sidx_v)
    plsc.subcore_barrier()

    def gat(j, b):
        return pltpu.make_async_copy(
            p_ref.at[sidx_v.at[pl.ds(j * PCHUNK, PCHUNK)]], rows_v.at[b],
            gsem.at[b])

    def dix(j, b):
        return pltpu.make_async_copy(
            dst_ref.at[pl.ds(e0 + j * PCHUNK, PCHUNK)], didx_v.at[b],
            gsem.at[b])

    def sca(b):
        return pltpu.make_async_copy(rows_v.at[b], agg.at[didx_v.at[b]],
                                     ssem.at[b])

    def prefetch(j, b):
        gat(j, b).start()
        dix(j, b).start()

    for b in range(PNBUF):
        prefetch(b, b)

    @pl.loop(0, PCHUNKS - PNBUF, step=PNBUF)
    def _(g):
        for b in range(PNBUF):
            gat(g + b, b).wait()
            dix(g + b, b).wait()
            pltpu.async_copy(rows_v.at[b], agg.at[didx_v.at[b]],
                             ssem.at[b], add=True)
        for b in range(PNBUF):
            sca(b).wait()
            prefetch(g + b + PNBUF, b)

    g0 = PCHUNKS - PNBUF
    for b in range(PNBUF):
        gat(g0 + b, b).wait()
        dix(g0 + b, b).wait()
        pltpu.async_copy(rows_v.at[b], agg.at[didx_v.at[b]],
                         ssem.at[b], add=True)
    for b in range(PNBUF):
        sca(b).wait()

    plsc.subcore_barrier()
    pltpu.sync_copy(agg.at[pl.ds(r0, ROWS_PER_TILE)],
                    out_ref.at[cid, pl.ds(r0, ROWS_PER_TILE)])


@functools.partial(jax.jit, static_argnames=("d",))
def _propagate(p, src, dst, d):
    zeros = jnp.zeros((NPAD, d), jnp.float32)
    cp = pltpu.CompilerParams(use_tc_tiling_on_sc=(d % 128 == 0))
    k = pl.kernel(
        _prop_body,
        out_type=jax.ShapeDtypeStruct((NC, NPAD, d), jnp.float32),
        mesh=_mesh,
        scratch_types=[
            pltpu.VMEM_SHARED((NPAD, d), jnp.float32),
            pltpu.VMEM((EPT,), jnp.int32),
            pltpu.VMEM((PNBUF, PCHUNK), jnp.int32),
            pltpu.VMEM((PNBUF, PCHUNK, d), jnp.float32),
            pltpu.SemaphoreType.DMA((PNBUF,)),
            pltpu.SemaphoreType.DMA((PNBUF,)),
        ],
        compiler_params=cp,
    )
    return k(p, src, dst, zeros)


RB = 2000
EB = 32000


def _split_body(ei_ref, src_ref, dst_ref):
    src_ref[...] = ei_ref[0]
    dst_ref[...] = ei_ref[1]


def _split(ei):
    src, dst = pl.pallas_call(
        _split_body,
        out_shape=[jax.ShapeDtypeStruct((E,), jnp.int32),
                   jax.ShapeDtypeStruct((E,), jnp.int32)],
    )(ei)
    return src, dst


def _hi_dot(a, b):
    return jnp.dot(a, b, preferred_element_type=jnp.float32,
                   precision=lax.Precision.HIGHEST)


def _matmul_body(x_ref, w_ref, o_ref):
    o_ref[...] = _hi_dot(x_ref[...], w_ref[...])


def _scale_body(p_ref, deg_ref, o_ref):
    norm = lax.rsqrt(jnp.maximum(deg_ref[...], 1.0))
    o_ref[...] = p_ref[...] * norm


def _mid_body(agg_ref, din_ref, dout_ref, b1_ref, w2_ref, o_ref):
    ndst = lax.rsqrt(jnp.maximum(din_ref[...], 1.0))
    nsrc = lax.rsqrt(jnp.maximum(dout_ref[...], 1.0))
    a = agg_ref[0] + agg_ref[1]
    h1 = jnp.maximum(a * ndst + b1_ref[...], 0.0)
    o_ref[...] = _hi_dot(h1 * nsrc, w2_ref[...])


def _post_body(agg_ref, din_ref, b2_ref, gid_ref,
               wc1_ref, bc1_ref, wc2_ref, bc2_ref, wc3_ref, bc3_ref,
               o_ref, sums_acc, counts_acc):
    i = pl.program_id(0)
    ndst = lax.rsqrt(jnp.maximum(din_ref[...], 1.0))
    a = agg_ref[0] + agg_ref[1]
    h2 = jnp.maximum(a * ndst + b2_ref[...], 0.0)
    gid = gid_ref[0]
    rows = lax.broadcasted_iota(jnp.int32, (G, RB), 0)
    onehot = (rows == gid).astype(jnp.float32)
    s = _hi_dot(onehot, h2)
    c = jnp.sum(onehot, axis=1, keepdims=True)

    @pl.when(i == 0)
    def _():
        sums_acc[...] = s
        counts_acc[...] = c

    @pl.when(i > 0)
    def _():
        sums_acc[...] += s
        counts_acc[...] += c

    @pl.when(i == N // RB - 1)
    def _():
        hg = sums_acc[...] / jnp.maximum(counts_acc[...], 1.0)
        out = _hi_dot(hg, wc1_ref[...]) + bc1_ref[...]
        out = _hi_dot(out, wc2_ref[...]) + bc2_ref[...]
        out = _hi_dot(out, wc3_ref[...]) + bc3_ref[...]
        o_ref[...] = out


def _row_spec(d):
    return pl.BlockSpec((RB, d), lambda i: (i, 0))


def _fix_spec(shape):
    nd = len(shape)
    return pl.BlockSpec(shape, lambda i: (0,) * nd)


def _matmul(x, W1):
    return pl.pallas_call(
        _matmul_body,
        grid=(N // RB,),
        in_specs=[_row_spec(x.shape[1]), _fix_spec(W1.shape)],
        out_specs=_row_spec(W1.shape[1]),
        out_shape=jax.ShapeDtypeStruct((N, W1.shape[1]), jnp.float32),
    )(x, W1)


def _scale(p, deg_out):
    d = p.shape[1]
    return pl.pallas_call(
        _scale_body,
        grid=(N // RB,),
        in_specs=[_row_spec(d), _row_spec(1)],
        out_specs=_row_spec(d),
        out_shape=jax.ShapeDtypeStruct((N, d), jnp.float32),
    )(p, deg_out)


def _mid(agg1, deg_in, deg_out, b1, W2):
    d = agg1.shape[2]
    return pl.pallas_call(
        _mid_body,
        grid=(N // RB,),
        in_specs=[pl.BlockSpec((NC, RB, d), lambda i: (0, i, 0)),
                  _row_spec(1), _row_spec(1),
                  _fix_spec((1, d)), _fix_spec(W2.shape)],
        out_specs=_row_spec(W2.shape[1]),
        out_shape=jax.ShapeDtypeStruct((N, W2.shape[1]), jnp.float32),
    )(agg1, deg_in, deg_out, b1, W2)


def _post(agg2, deg_in, b2, gid, Wc1, bc1, Wc2, bc2, Wc3, bc3):
    d = agg2.shape[2]
    return pl.pallas_call(
        _post_body,
        grid=(N // RB,),
        in_specs=[pl.BlockSpec((NC, RB, d), lambda i: (0, i, 0)),
                  _row_spec(1),
                  _fix_spec((1, d)),
                  pl.BlockSpec((1, 1, RB), lambda i: (i, 0, 0)),
                  _fix_spec(Wc1.shape), _fix_spec((1, Wc1.shape[1])),
                  _fix_spec(Wc2.shape), _fix_spec((1, Wc2.shape[1])),
                  _fix_spec(Wc3.shape), _fix_spec((1, Wc3.shape[1]))],
        out_specs=_fix_spec((G, Wc3.shape[1])),
        out_shape=jax.ShapeDtypeStruct((G, Wc3.shape[1]), jnp.float32),
        scratch_shapes=[pltpu.VMEM((G, d), jnp.float32),
                        pltpu.VMEM((G, 1), jnp.float32)],
    )(agg2, deg_in, b2, gid, Wc1, bc1, Wc2, bc2, Wc3, bc3)


def kernel(x, edge_index, graph_ids, W1, b1, W2, b2,
           Wc1, bc1, Wc2, bc2, Wc3, bc3):
    ei = edge_index.astype(jnp.int32)
    src, dst = _split(ei)

    degs = _degrees(src, dst)
    deg_out = degs[0, :N].reshape(N, 1)
    deg_in = degs[1, :N].reshape(N, 1)

    p1u = _matmul(x, W1)
    p1 = _scale(p1u, deg_out)
    agg1 = _propagate(p1, src, dst, W1.shape[1])
    p2 = _mid(agg1, deg_in, deg_out, b1.reshape(1, -1), W2)
    agg2 = _propagate(p2, src, dst, W2.shape[1])
    out = _post(agg2, deg_in, b2.reshape(1, -1),
                graph_ids.astype(jnp.int32).reshape(N // RB, 1, RB),
                Wc1, bc1.reshape(1, -1), Wc2, bc2.reshape(1, -1),
                Wc3, bc3.reshape(1, -1))
    return out


# speedup vs baseline: 3.2848x; 1.0183x over previous
import functools
import jax
import jax.numpy as jnp
from jax import lax
from jax.experimental import pallas as pl
from jax.experimental.pallas import tpu as pltpu
from jax.experimental.pallas import tpu_sc as plsc

N = 10000
E = 320000
G = 64
NPAD = 10240
NC, NS = 2, 16
NW = NC * NS
ROWS_PER_TILE = NPAD // NS
CHUNK = 128
EPT = E // NW
PCHUNK = 40
PCHUNKS = EPT // PCHUNK
PNBUF = 5
DEPT = E // NS
DFULL = DEPT // CHUNK
DTAIL = DEPT - DFULL * CHUNK
DW = 12

_mesh = plsc.VectorSubcoreMesh(core_axis_name="c", subcore_axis_name="s")


def _deg_body(src_ref, dst_ref, ones_ref, zeros_ref, out_ref,
              hist, ones_v, idx_v, tidx_v, gsem, ssem):
    cid = lax.axis_index("c")
    sid = lax.axis_index("s")
    r0 = sid * ROWS_PER_TILE
    pltpu.sync_copy(zeros_ref.at[pl.ds(r0, ROWS_PER_TILE)],
                    hist.at[pl.ds(r0, ROWS_PER_TILE)])
    pltpu.sync_copy(ones_ref, ones_v)
    plsc.subcore_barrier()

    e0 = sid * DEPT

    def run(e_ref):
        def idx(j, b):
            return pltpu.make_async_copy(
                e_ref.at[pl.ds(e0 + j * CHUNK, CHUNK)], idx_v.at[b],
                gsem.at[b])

        def sca(b):
            return pltpu.make_async_copy(ones_v, hist.at[idx_v.at[b]],
                                         ssem.at[b])

        for b in range(DW):
            idx(b, b).start()

        @pl.loop(0, DFULL - DW, step=DW)
        def _(g):
            for b in range(DW):
                idx(g + b, b).wait()
                pltpu.async_copy(ones_v, hist.at[idx_v.at[b]], ssem.at[b],
                                 add=True)
            for b in range(DW):
                sca(b).wait()
                idx(g + b + DW, b).start()

        g0 = DFULL - DW
        for b in range(DW):
            idx(g0 + b, b).wait()
            pltpu.async_copy(ones_v, hist.at[idx_v.at[b]], ssem.at[b],
                             add=True)
        for b in range(DW):
            sca(b).wait()

        pltpu.sync_copy(e_ref.at[pl.ds(e0 + DFULL * CHUNK, DTAIL)], tidx_v)
        pltpu.sync_copy(ones_v.at[pl.ds(0, DTAIL)], hist.at[tidx_v],
                        add=True)

    @pl.when(cid == 0)
    def _():
        run(src_ref)

    @pl.when(cid == 1)
    def _():
        run(dst_ref)

    plsc.subcore_barrier()
    pltpu.sync_copy(hist.at[pl.ds(r0, ROWS_PER_TILE)],
                    out_ref.at[cid, pl.ds(r0, ROWS_PER_TILE)])


@jax.jit
def _degrees(src, dst):
    ones = jnp.ones((CHUNK,), jnp.float32)
    zeros = jnp.zeros((NPAD,), jnp.float32)
    k = pl.kernel(
        _deg_body,
        out_type=jax.ShapeDtypeStruct((NC, NPAD), jnp.float32),
        mesh=_mesh,
        scratch_types=[
            pltpu.VMEM_SHARED((NPAD,), jnp.float32),
            pltpu.VMEM((CHUNK,), jnp.float32),
            pltpu.VMEM((DW, CHUNK), jnp.int32),
            pltpu.VMEM((DTAIL,), jnp.int32),
            pltpu.SemaphoreType.DMA((DW,)),
            pltpu.SemaphoreType.DMA((DW,)),
        ],
    )
    return k(src, dst, ones, zeros)


def _prop_body(pchunk, p_ref, src_ref, dst_ref, zeros_ref, out_ref,
               agg, sidx_v, didx_v, rows_v, gsem, ssem):
    n_chunks = EPT // pchunk
    cid = lax.axis_index("c")
    sid = lax.axis_index("s")
    r0 = sid * ROWS_PER_TILE
    pltpu.sync_copy(zeros_ref.at[pl.ds(r0, ROWS_PER_TILE)],
                    agg.at[pl.ds(r0, ROWS_PER_TILE)])
    wid = cid * NS + sid
    e0 = wid * EPT
    pltpu.sync_copy(src_ref.at[pl.ds(e0, EPT)], sidx_v)
    plsc.subcore_barrier()

    def gat(j, b):
        return pltpu.make_async_copy(
            p_ref.at[sidx_v.at[pl.ds(j * pchunk, pchunk)]], rows_v.at[b],
            gsem.at[b])

    def dix(j, b):
        return pltpu.make_async_copy(
            dst_ref.at[pl.ds(e0 + j * pchunk, pchunk)], didx_v.at[b],
            gsem.at[b])

    def sca(b):
        return pltpu.make_async_copy(rows_v.at[b], agg.at[didx_v.at[b]],
                                     ssem.at[b])

    def prefetch(j, b):
        gat(j, b).start()
        dix(j, b).start()

    for b in range(PNBUF):
        prefetch(b, b)

    @pl.loop(0, n_chunks - PNBUF, step=PNBUF)
    def _(g):
        for b in range(PNBUF):
            gat(g + b, b).wait()
            dix(g + b, b).wait()
            pltpu.async_copy(rows_v.at[b], agg.at[didx_v.at[b]],
                             ssem.at[b], add=True)
        for b in range(PNBUF):
            sca(b).wait()
            prefetch(g + b + PNBUF, b)

    g0 = n_chunks - PNBUF
    for b in range(PNBUF):
        gat(g0 + b, b).wait()
        dix(g0 + b, b).wait()
        pltpu.async_copy(rows_v.at[b], agg.at[didx_v.at[b]],
                         ssem.at[b], add=True)
    for b in range(PNBUF):
        sca(b).wait()

    plsc.subcore_barrier()
    pltpu.sync_copy(agg.at[pl.ds(r0, ROWS_PER_TILE)],
                    out_ref.at[cid, pl.ds(r0, ROWS_PER_TILE)])


@functools.partial(jax.jit, static_argnames=("d",))
def _propagate(p, src, dst, d):
    zeros = jnp.zeros((NPAD, d), jnp.float32)
    cp = pltpu.CompilerParams(use_tc_tiling_on_sc=(d % 128 == 0))
    pchunk = PCHUNK if d > 64 else 2 * PCHUNK
    k = pl.kernel(
        functools.partial(_prop_body, pchunk),
        out_type=jax.ShapeDtypeStruct((NC, NPAD, d), jnp.float32),
        mesh=_mesh,
        scratch_types=[
            pltpu.VMEM_SHARED((NPAD, d), jnp.float32),
            pltpu.VMEM((EPT,), jnp.int32),
            pltpu.VMEM((PNBUF, pchunk), jnp.int32),
            pltpu.VMEM((PNBUF, pchunk, d), jnp.float32),
            pltpu.SemaphoreType.DMA((PNBUF,)),
            pltpu.SemaphoreType.DMA((PNBUF,)),
        ],
        compiler_params=cp,
    )
    return k(p, src, dst, zeros)


RB = 2000
EB = 32000


def _split_body(ei_ref, src_ref, dst_ref):
    src_ref[...] = ei_ref[0]
    dst_ref[...] = ei_ref[1]


def _split(ei):
    src, dst = pl.pallas_call(
        _split_body,
        out_shape=[jax.ShapeDtypeStruct((E,), jnp.int32),
                   jax.ShapeDtypeStruct((E,), jnp.int32)],
    )(ei)
    return src, dst


def _hi_dot(a, b):
    return jnp.dot(a, b, preferred_element_type=jnp.float32,
                   precision=lax.Precision.HIGHEST)


def _matmul_body(x_ref, w_ref, o_ref):
    o_ref[...] = _hi_dot(x_ref[...], w_ref[...])


def _scale_body(p_ref, deg_ref, o_ref):
    norm = lax.rsqrt(jnp.maximum(deg_ref[...], 1.0))
    o_ref[...] = p_ref[...] * norm


def _mid_body(agg_ref, din_ref, dout_ref, b1_ref, w2_ref, o_ref):
    ndst = lax.rsqrt(jnp.maximum(din_ref[...], 1.0))
    nsrc = lax.rsqrt(jnp.maximum(dout_ref[...], 1.0))
    a = agg_ref[0] + agg_ref[1]
    h1 = jnp.maximum(a * ndst + b1_ref[...], 0.0)
    o_ref[...] = _hi_dot(h1 * nsrc, w2_ref[...])


def _tdot(a, b):
    return lax.dot_general(a, b, (((0,), (0,)), ((), ())),
                           preferred_element_type=jnp.float32,
                           precision=lax.Precision.HIGHEST)


def _post_body(agg_ref, din_ref, b2_ref, gid_ref,
               wc1_ref, bc1_ref, wc2_ref, bc2_ref, wc3_ref, bc3_ref,
               o_ref, sums_acc, counts_acc):
    i = pl.program_id(0)
    ndst = lax.rsqrt(jnp.maximum(din_ref[...], 1.0))
    a = agg_ref[0] + agg_ref[1]
    h2 = jnp.maximum(a * ndst + b2_ref[...], 0.0)
    gid = gid_ref[...]
    cols = lax.broadcasted_iota(jnp.int32, (RB, G), 1)
    onehot = (cols == gid).astype(jnp.float32)
    s = _tdot(onehot, h2)
    c = _tdot(onehot, jnp.ones((RB, 1), jnp.float32))

    @pl.when(i == 0)
    def _():
        sums_acc[...] = s
        counts_acc[...] = c

    @pl.when(i > 0)
    def _():
        sums_acc[...] += s
        counts_acc[...] += c

    @pl.when(i == N // RB - 1)
    def _():
        hg = sums_acc[...] / jnp.maximum(counts_acc[...], 1.0)
        out = _hi_dot(hg, wc1_ref[...]) + bc1_ref[...]
        out = _hi_dot(out, wc2_ref[...]) + bc2_ref[...]
        out = _hi_dot(out, wc3_ref[...]) + bc3_ref[...]
        o_ref[...] = out


def _row_spec(d):
    return pl.BlockSpec((RB, d), lambda i: (i, 0))


def _fix_spec(shape):
    nd = len(shape)
    return pl.BlockSpec(shape, lambda i: (0,) * nd)


def _matmul(x, W1):
    return pl.pallas_call(
        _matmul_body,
        grid=(N // RB,),
        in_specs=[_row_spec(x.shape[1]), _fix_spec(W1.shape)],
        out_specs=_row_spec(W1.shape[1]),
        out_shape=jax.ShapeDtypeStruct((N, W1.shape[1]), jnp.float32),
    )(x, W1)


def _scale(p, deg_out):
    d = p.shape[1]
    return pl.pallas_call(
        _scale_body,
        grid=(N // RB,),
        in_specs=[_row_spec(d), _row_spec(1)],
        out_specs=_row_spec(d),
        out_shape=jax.ShapeDtypeStruct((N, d), jnp.float32),
    )(p, deg_out)


def _mid(agg1, deg_in, deg_out, b1, W2):
    d = agg1.shape[2]
    return pl.pallas_call(
        _mid_body,
        grid=(N // RB,),
        in_specs=[pl.BlockSpec((NC, RB, d), lambda i: (0, i, 0)),
                  _row_spec(1), _row_spec(1),
                  _fix_spec((1, d)), _fix_spec(W2.shape)],
        out_specs=_row_spec(W2.shape[1]),
        out_shape=jax.ShapeDtypeStruct((N, W2.shape[1]), jnp.float32),
    )(agg1, deg_in, deg_out, b1, W2)


def _post(agg2, deg_in, b2, gid, Wc1, bc1, Wc2, bc2, Wc3, bc3):
    d = agg2.shape[2]
    return pl.pallas_call(
        _post_body,
        grid=(N // RB,),
        in_specs=[pl.BlockSpec((NC, RB, d), lambda i: (0, i, 0)),
                  _row_spec(1),
                  _fix_spec((1, d)),
                  _row_spec(1),
                  _fix_spec(Wc1.shape), _fix_spec((1, Wc1.shape[1])),
                  _fix_spec(Wc2.shape), _fix_spec((1, Wc2.shape[1])),
                  _fix_spec(Wc3.shape), _fix_spec((1, Wc3.shape[1]))],
        out_specs=_fix_spec((G, Wc3.shape[1])),
        out_shape=jax.ShapeDtypeStruct((G, Wc3.shape[1]), jnp.float32),
        scratch_shapes=[pltpu.VMEM((G, d), jnp.float32),
                        pltpu.VMEM((G, 1), jnp.float32)],
    )(agg2, deg_in, b2, gid, Wc1, bc1, Wc2, bc2, Wc3, bc3)


def kernel(x, edge_index, graph_ids, W1, b1, W2, b2,
           Wc1, bc1, Wc2, bc2, Wc3, bc3):
    ei = edge_index.astype(jnp.int32)
    src, dst = _split(ei)

    degs = _degrees(src, dst)
    deg_out = degs[0, :N].reshape(N, 1)
    deg_in = degs[1, :N].reshape(N, 1)

    p1u = _matmul(x, W1)
    p1 = _scale(p1u, deg_out)
    agg1 = _propagate(p1, src, dst, W1.shape[1])
    p2 = _mid(agg1, deg_in, deg_out, b1.reshape(1, -1), W2)
    agg2 = _propagate(p2, src, dst, W2.shape[1])
    out = _post(agg2, deg_in, b2.reshape(1, -1),
                graph_ids.astype(jnp.int32).reshape(N, 1),
                Wc1, bc1.reshape(1, -1), Wc2, bc2.reshape(1, -1),
                Wc3, bc3.reshape(1, -1))
    return out
